# baseline (speedup 1.0000x reference)
import functools

import jax
import jax.numpy as jnp
from jax import lax
from jax.experimental import pallas as pl
from jax.experimental.pallas import tpu as pltpu

FEATURE_SIZE = 256
EMB = 256
HEADS = 4
HEAD_DIM = EMB // HEADS
EDGE_DIM = 16
N_LAYERS = 3
N_GRAPHS = 256
N_RAW = 10000
N_TREE = 4000
INV_SQRT_D = 1.0 / (HEAD_DIM ** 0.5)
LOGIT_CLAMP = 70.0


def _proj_body(x_ref, wq_ref, wk_ref, wv_ref, q_ref, k_ref, v_ref):
    x = x_ref[...]
    q_ref[...] = jnp.dot(x, wq_ref[...], preferred_element_type=jnp.float32)
    k_ref[...] = jnp.dot(x, wk_ref[...], preferred_element_type=jnp.float32)
    v_ref[...] = jnp.dot(x, wv_ref[...], preferred_element_type=jnp.float32)


def _proj_qkv(x, wq, wk, wv, bm):
    n = x.shape[0]
    grid = (n // bm,)
    wspec = pl.BlockSpec((EMB, EMB), lambda i: (0, 0))
    rspec = pl.BlockSpec((bm, EMB), lambda i: (i, 0))
    return pl.pallas_call(
        _proj_body,
        grid=grid,
        in_specs=[rspec, wspec, wspec, wspec],
        out_specs=[rspec, rspec, rspec],
        out_shape=[jax.ShapeDtypeStruct((n, EMB), jnp.float32)] * 3,
    )(x, wq, wk, wv)


def _eproj_body(ea_ref, we_ref, e_ref):
    e_ref[...] = jnp.dot(ea_ref[...], we_ref[...],
                         preferred_element_type=jnp.float32)


def _proj_e(edge_attr, we, bm):
    m = edge_attr.shape[0]
    grid = (m // bm,)
    return pl.pallas_call(
        _eproj_body,
        grid=grid,
        in_specs=[pl.BlockSpec((bm, EDGE_DIM), lambda i: (i, 0)),
                  pl.BlockSpec((EDGE_DIM, EMB), lambda i: (0, 0))],
        out_specs=pl.BlockSpec((bm, EMB), lambda i: (i, 0)),
        out_shape=jax.ShapeDtypeStruct((m, EMB), jnp.float32),
    )(edge_attr, we)


def _update_body(agg_ref, x_ref, wo_ref, ws_ref, o_ref):
    acc = jnp.dot(agg_ref[...], wo_ref[...], preferred_element_type=jnp.float32)
    acc += jnp.dot(x_ref[...], ws_ref[...], preferred_element_type=jnp.float32)
    o_ref[...] = jnp.maximum(acc, 0.0)


def _update(agg, x, wo, ws, bm):
    n = x.shape[0]
    grid = (n // bm,)
    wspec = pl.BlockSpec((EMB, EMB), lambda i: (0, 0))
    rspec = pl.BlockSpec((bm, EMB), lambda i: (i, 0))
    return pl.pallas_call(
        _update_body,
        grid=grid,
        in_specs=[rspec, rspec, wspec, wspec],
        out_specs=rspec,
        out_shape=jax.ShapeDtypeStruct((n, EMB), jnp.float32),
    )(agg, x, wo, ws)


def _pool_body(starts_ref, x_ref, out_ref):
    g = pl.program_id(0)
    start = starts_ref[g]
    end = starts_ref[g + 1]
    c0 = start // 8
    c1 = lax.select(end > start, (end - 1) // 8 + 1, c0)

    def body(c, carry):
        mx, sm = carry
        rows = x_ref[pl.ds(c * 8, 8), :]
        rid = c * 8 + lax.broadcasted_iota(jnp.int32, (8, 1), 0)
        m = jnp.logical_and(rid >= start, rid < end)
        mx = jnp.maximum(mx, jnp.where(m, rows, -jnp.inf))
        sm = sm + jnp.where(m, rows, 0.0)
        return mx, sm

    mx0 = jnp.full((8, EMB), -jnp.inf, jnp.float32)
    sm0 = jnp.zeros((8, EMB), jnp.float32)
    mx, sm = lax.fori_loop(c0, c1, body, (mx0, sm0))
    mxr = jnp.max(mx, axis=0, keepdims=True)
    mxr = jnp.where(jnp.isfinite(mxr), mxr, 0.0)
    smr = jnp.sum(sm, axis=0, keepdims=True)
    cnt = (end - start).astype(jnp.float32)
    mean = smr / (cnt + 1e-16)
    out_ref[0, :, :EMB] = mxr
    out_ref[0, :, EMB:] = mean


def _pool(x, starts):
    n = x.shape[0]
    out = pl.pallas_call(
        _pool_body,
        grid=(N_GRAPHS,),
        in_specs=[pl.BlockSpec(memory_space=pltpu.SMEM),
                  pl.BlockSpec((n, EMB), lambda g: (0, 0))],
        out_specs=pl.BlockSpec((1, 1, 2 * EMB), lambda g: (g, 0, 0)),
        out_shape=jax.ShapeDtypeStruct((N_GRAPHS, 1, 2 * EMB), jnp.float32),
    )(starts, x)
    return out.reshape(N_GRAPHS, 2 * EMB)


def _head_body(pt_ref, pr_ref, wt_ref, wr_ref, bl_ref, lg_ref, lb_ref,
               w1_ref, b1_ref, w2_ref, b2_ref, y_ref):
    x = jnp.dot(pt_ref[...], wt_ref[...], preferred_element_type=jnp.float32)
    x += jnp.dot(pr_ref[...], wr_ref[...], preferred_element_type=jnp.float32)
    x += bl_ref[...]
    mu = jnp.mean(x, axis=-1, keepdims=True)
    var = jnp.mean((x - mu) ** 2, axis=-1, keepdims=True)
    x = (x - mu) / jnp.sqrt(var + 1e-5) * lg_ref[...] + lb_ref[...]
    h = jnp.maximum(x, 0.0)
    h = jnp.maximum(jnp.dot(h, w1_ref[...], preferred_element_type=jnp.float32)
                    + b1_ref[...], 0.0)
    y_ref[...] = jnp.dot(h, w2_ref[...], preferred_element_type=jnp.float32) \
        + b2_ref[...]


def _head(pooled_t, pooled_r, params):
    wt = params['W_lin'][: 2 * EMB]
    wr = params['W_lin'][2 * EMB:]
    vm = lambda shape: pl.BlockSpec(shape, lambda: tuple(0 for _ in shape))
    return pl.pallas_call(
        _head_body,
        in_specs=[vm(pooled_t.shape), vm(pooled_r.shape), vm(wt.shape),
                  vm(wr.shape), vm(params['b_lin'].shape),
                  vm(params['ln_g'].shape), vm(params['ln_b'].shape),
                  vm(params['W1'].shape), vm(params['b1'].shape),
                  vm(params['W2'].shape), vm(params['b2'].shape)],
        out_specs=vm((N_GRAPHS, 1)),
        out_shape=jax.ShapeDtypeStruct((N_GRAPHS, 1), jnp.float32),
    )(pooled_t, pooled_r, wt, wr, params['b_lin'], params['ln_g'],
      params['ln_b'], params['W1'], params['b1'], params['W2'], params['b2'])


def _edge_phase(q, k, v, e, src, dst, n_nodes):
    qh = q.reshape(n_nodes, HEADS, HEAD_DIM)
    kh = k.reshape(n_nodes, HEADS, HEAD_DIM)
    vh = v.reshape(n_nodes, HEADS, HEAD_DIM)
    eh = e.reshape(-1, HEADS, HEAD_DIM)
    k_e = kh[src] + eh
    v_e = vh[src] + eh
    logits = (qh[dst] * k_e).sum(-1) * INV_SQRT_D
    ex = jnp.exp(jnp.minimum(logits, LOGIT_CLAMP))
    s = jax.ops.segment_sum(ex, dst, num_segments=n_nodes)
    alpha = ex / (s[dst] + 1e-16)
    msg = alpha[..., None] * v_e
    agg = jax.ops.segment_sum(msg, dst, num_segments=n_nodes)
    return agg.reshape(n_nodes, EMB), alpha


def _encoder(x, edge_attr, edge_index, starts, p, n_nodes, bm_n, bm_e):
    src = edge_index[0]
    dst = edge_index[1]
    alpha = None
    for l in range(N_LAYERS):
        q, k, v = _proj_qkv(x, p['Wq'][l], p['Wk'][l], p['Wv'][l], bm_n)
        e = _proj_e(edge_attr, p['We'][l], bm_e)
        agg, alpha = _edge_phase(q, k, v, e, src, dst, n_nodes)
        x = _update(agg, x, p['Wo'][l], p['Ws'][l], bm_n)
    pooled = _pool(x, starts)
    att_mean = alpha.mean(axis=-1)
    return pooled, att_mean


def kernel(x_raw, edge_attr_raw, x_tree, edge_attr_tree, params,
           edge_index_raw, batch_raw, edge_index_tree, batch_tree):
    gids = jnp.arange(N_GRAPHS + 1, dtype=jnp.int32)
    starts_r = jnp.searchsorted(batch_raw, gids, side='left').astype(jnp.int32)
    starts_t = jnp.searchsorted(batch_tree, gids, side='left').astype(jnp.int32)

    pooled_r, att_r = _encoder(x_raw, edge_attr_raw, edge_index_raw, starts_r,
                               params['enc_raw'], N_RAW, 1000, 4000)
    pooled_t, att_t = _encoder(x_tree, edge_attr_tree, edge_index_tree,
                               starts_t, params['enc_tree'], N_TREE, 1000, 4000)
    y = _head(pooled_t, pooled_r, params)
    return (y, edge_index_raw, att_r, edge_index_tree, att_t)


# speedup vs baseline: 4.2464x; 4.2464x over previous
import functools

import jax
import jax.numpy as jnp
from jax import lax
from jax.experimental import pallas as pl
from jax.experimental.pallas import tpu as pltpu
from jax.experimental.pallas import tpu_sc as plsc

EMB = 256
HEADS = 4
HEAD_DIM = EMB // HEADS
EDGE_DIM = 16
N_LAYERS = 3
N_GRAPHS = 256
N_RAW = 10000
N_TREE = 4000
INV_SQRT_D = 1.0 / (HEAD_DIM ** 0.5)
LOGIT_CLAMP = 70.0

NW = 32
EC = 128


def _proj_body(x_ref, wq_ref, wk_ref, wv_ref, q_ref, k_ref, v_ref):
    x = x_ref[...]
    q_ref[...] = jnp.dot(x, wq_ref[...], preferred_element_type=jnp.float32)
    k_ref[...] = jnp.dot(x, wk_ref[...], preferred_element_type=jnp.float32)
    v_ref[...] = jnp.dot(x, wv_ref[...], preferred_element_type=jnp.float32)


def _proj_qkv(x, wq, wk, wv, bm):
    n = x.shape[0]
    grid = (n // bm,)
    wspec = pl.BlockSpec((EMB, EMB), lambda i: (0, 0))
    rspec = pl.BlockSpec((bm, EMB), lambda i: (i, 0))
    return pl.pallas_call(
        _proj_body,
        grid=grid,
        in_specs=[rspec, wspec, wspec, wspec],
        out_specs=[rspec, rspec, rspec],
        out_shape=[jax.ShapeDtypeStruct((n, EMB), jnp.float32)] * 3,
    )(x, wq, wk, wv)


def _eproj_body(ea_ref, we_ref, e_ref):
    e_ref[...] = jnp.dot(ea_ref[...], we_ref[...],
                         preferred_element_type=jnp.float32)


def _proj_e(edge_attr, we, bm):
    m = edge_attr.shape[0]
    grid = (m // bm,)
    return pl.pallas_call(
        _eproj_body,
        grid=grid,
        in_specs=[pl.BlockSpec((bm, EDGE_DIM), lambda i: (i, 0)),
                  pl.BlockSpec((EDGE_DIM, EMB), lambda i: (0, 0))],
        out_specs=pl.BlockSpec((bm, EMB), lambda i: (i, 0)),
        out_shape=jax.ShapeDtypeStruct((m, EMB), jnp.float32),
    )(edge_attr, we)


def _rec_body(s2_ref, rec_ref):
    rec_ref[...] = 1.0 / (s2_ref[0] + s2_ref[1] + 1e-16)


def _srecip(s2, n_pad):
    return pl.pallas_call(
        _rec_body,
        in_specs=[pl.BlockSpec((2, n_pad, HEADS), lambda: (0, 0, 0))],
        out_specs=pl.BlockSpec((n_pad, HEADS), lambda: (0, 0)),
        out_shape=jax.ShapeDtypeStruct((n_pad, HEADS), jnp.float32),
    )(s2)


def _update_body(agg2_ref, rec_ref, x_ref, wo_ref, ws_ref, o_ref):
    acc = jnp.dot(x_ref[...], ws_ref[...], preferred_element_type=jnp.float32)
    for h in range(HEADS):
        half = agg2_ref[h // 2]
        col = (h % 2) * HEAD_DIM
        scaled = half[:, col:col + HEAD_DIM] * rec_ref[:, h:h + 1]
        acc += jnp.dot(scaled, wo_ref[h * HEAD_DIM:(h + 1) * HEAD_DIM, :],
                       preferred_element_type=jnp.float32)
    o_ref[...] = jnp.maximum(acc, 0.0)


def _update(agg2, rec, x, wo, ws, bm):
    n = x.shape[0]
    grid = (n // bm,)
    wspec = pl.BlockSpec((EMB, EMB), lambda i: (0, 0))
    rspec = pl.BlockSpec((bm, EMB), lambda i: (i, 0))
    return pl.pallas_call(
        _update_body,
        grid=grid,
        in_specs=[pl.BlockSpec((2, bm, EMB // 2), lambda i: (0, i, 0)),
                  pl.BlockSpec((bm, HEADS), lambda i: (i, 0)),
                  rspec, wspec, wspec],
        out_specs=rspec,
        out_shape=jax.ShapeDtypeStruct((n, EMB), jnp.float32),
    )(agg2, rec, x, wo, ws)


def _pool_body(starts_ref, x_ref, out_ref):
    g = pl.program_id(0)
    start = starts_ref[g]
    end = starts_ref[g + 1]
    c0 = start // 8
    c1 = lax.select(end > start, (end - 1) // 8 + 1, c0)

    def body(c, carry):
        mx, sm = carry
        rows = x_ref[pl.ds(c * 8, 8), :]
        rid = c * 8 + lax.broadcasted_iota(jnp.int32, (8, 1), 0)
        m = jnp.logical_and(rid >= start, rid < end)
        mx = jnp.maximum(mx, jnp.where(m, rows, -jnp.inf))
        sm = sm + jnp.where(m, rows, 0.0)
        return mx, sm

    mx0 = jnp.full((8, EMB), -jnp.inf, jnp.float32)
    sm0 = jnp.zeros((8, EMB), jnp.float32)
    mx, sm = lax.fori_loop(c0, c1, body, (mx0, sm0))
    mxr = jnp.max(mx, axis=0, keepdims=True)
    mxr = jnp.where(jnp.isfinite(mxr), mxr, 0.0)
    smr = jnp.sum(sm, axis=0, keepdims=True)
    cnt = (end - start).astype(jnp.float32)
    mean = smr / (cnt + 1e-16)
    out_ref[0, :, :EMB] = mxr
    out_ref[0, :, EMB:] = mean


def _pool(x, starts, n_real):
    out = pl.pallas_call(
        _pool_body,
        grid=(N_GRAPHS,),
        in_specs=[pl.BlockSpec(memory_space=pltpu.SMEM),
                  pl.BlockSpec((n_real, EMB), lambda g: (0, 0))],
        out_specs=pl.BlockSpec((1, 1, 2 * EMB), lambda g: (g, 0, 0)),
        out_shape=jax.ShapeDtypeStruct((N_GRAPHS, 1, 2 * EMB), jnp.float32),
    )(starts, x)
    return out.reshape(N_GRAPHS, 2 * EMB)


def _head_body(pt_ref, pr_ref, wt_ref, wr_ref, bl_ref, lg_ref, lb_ref,
               w1_ref, b1_ref, w2_ref, b2_ref, y_ref):
    x = jnp.dot(pt_ref[...], wt_ref[...], preferred_element_type=jnp.float32)
    x += jnp.dot(pr_ref[...], wr_ref[...], preferred_element_type=jnp.float32)
    x += bl_ref[...]
    mu = jnp.mean(x, axis=-1, keepdims=True)
    var = jnp.mean((x - mu) ** 2, axis=-1, keepdims=True)
    x = (x - mu) / jnp.sqrt(var + 1e-5) * lg_ref[...] + lb_ref[...]
    h = jnp.maximum(x, 0.0)
    h = jnp.maximum(jnp.dot(h, w1_ref[...], preferred_element_type=jnp.float32)
                    + b1_ref[...], 0.0)
    y_ref[...] = jnp.dot(h, w2_ref[...], preferred_element_type=jnp.float32) \
        + b2_ref[...]


def _head(pooled_t, pooled_r, params):
    wt = params['W_lin'][: 2 * EMB]
    wr = params['W_lin'][2 * EMB:]
    vm = lambda shape: pl.BlockSpec(shape, lambda: tuple(0 for _ in shape))
    return pl.pallas_call(
        _head_body,
        in_specs=[vm(pooled_t.shape), vm(pooled_r.shape), vm(wt.shape),
                  vm(wr.shape), vm(params['b_lin'].shape),
                  vm(params['ln_g'].shape), vm(params['ln_b'].shape),
                  vm(params['W1'].shape), vm(params['b1'].shape),
                  vm(params['W2'].shape), vm(params['b2'].shape)],
        out_specs=vm((N_GRAPHS, 1)),
        out_shape=jax.ShapeDtypeStruct((N_GRAPHS, 1), jnp.float32),
    )(pooled_t, pooled_r, wt, wr, params['b_lin'], params['ln_g'],
      params['ln_b'], params['W1'], params['b1'], params['W2'], params['b2'])


def _make_scatter4(n_pad, e_pad):
    ew = e_pad // NW
    steps = ew // EC
    mesh = plsc.VectorSubcoreMesh(core_axis_name="c", subcore_axis_name="s",
                                  num_cores=2, num_subcores=16)

    @functools.partial(
        pl.kernel, mesh=mesh,
        out_type=jax.ShapeDtypeStruct((2 * n_pad, HEADS), jnp.float32),
        scratch_types=[
            pltpu.VMEM((EC,), jnp.int32),
            pltpu.VMEM((EC, HEADS), jnp.float32),
            pltpu.VMEM_SHARED((n_pad, HEADS), jnp.float32),
        ])
    def scatter4(ex_hbm, dst_hbm, z4_hbm, s2_hbm, dstv, buf, s_sh):
        c = lax.axis_index("c")
        sid = lax.axis_index("s")
        w = sid * 2 + c

        @pl.when(sid == 0)
        def _():
            pltpu.sync_copy(z4_hbm, s_sh)
        plsc.subcore_barrier()

        def step(i, _):
            base = w * ew + i * EC
            pltpu.sync_copy(dst_hbm.at[pl.ds(base, EC)], dstv)
            pltpu.sync_copy(ex_hbm.at[pl.ds(base, EC), :], buf)
            pltpu.sync_copy(buf, s_sh.at[dstv], add=True)
            return 0

        lax.fori_loop(0, steps, step, 0, unroll=False)
        plsc.subcore_barrier()
        rpt = n_pad // 16
        pltpu.sync_copy(s_sh.at[pl.ds(sid * rpt, rpt), :],
                        s2_hbm.at[pl.ds(c * n_pad + sid * rpt, rpt), :])

    return scatter4


def _make_scatter128(n_pad, e_pad):
    ew = e_pad // 16
    steps = ew // EC
    mesh = plsc.VectorSubcoreMesh(core_axis_name="c", subcore_axis_name="s",
                                  num_cores=2, num_subcores=16)

    @functools.partial(
        pl.kernel, mesh=mesh,
        out_type=jax.ShapeDtypeStruct((2 * n_pad, EMB // 2), jnp.float32),
        scratch_types=[
            pltpu.VMEM((EC,), jnp.int32),
            pltpu.VMEM((EC, EMB // 2), jnp.float32),
            pltpu.VMEM_SHARED((n_pad, EMB // 2), jnp.float32),
        ])
    def scatter128(msgh_hbm, dst_hbm, z128_hbm, agg2_hbm, dstv, buf, a_sh):
        c = lax.axis_index("c")
        sid = lax.axis_index("s")

        @pl.when(sid == 0)
        def _():
            pltpu.sync_copy(z128_hbm, a_sh)
        plsc.subcore_barrier()

        def step(i, _):
            base = sid * ew + i * EC
            pltpu.sync_copy(dst_hbm.at[pl.ds(base, EC)], dstv)
            pltpu.sync_copy(msgh_hbm.at[pl.ds(c * e_pad + base, EC), :], buf)
            pltpu.sync_copy(buf, a_sh.at[dstv], add=True)
            return 0

        lax.fori_loop(0, steps, step, 0, unroll=False)
        plsc.subcore_barrier()
        rpt = n_pad // 16
        pltpu.sync_copy(a_sh.at[pl.ds(sid * rpt, rpt), :],
                        agg2_hbm.at[pl.ds(c * n_pad + sid * rpt, rpt), :])

    return scatter128


def _encoder(x, edge_attr, edge_index, starts, p, n_real, n_pad, e_pad,
             bm_n, bm_e):
    e_real = edge_index.shape[1]
    src = jnp.concatenate([edge_index[0],
                           jnp.zeros((e_pad - e_real,), jnp.int32)])
    dst = jnp.concatenate([edge_index[1],
                           jnp.full((e_pad - e_real,), n_real, jnp.int32)])
    ea = jnp.zeros((e_pad, EDGE_DIM), jnp.float32).at[:e_real].set(edge_attr)
    xp = jnp.zeros((n_pad, EMB), jnp.float32).at[:n_real].set(x)

    z4 = jnp.zeros((n_pad, HEADS), jnp.float32)
    z128 = jnp.zeros((n_pad, EMB // 2), jnp.float32)

    scatter4 = _make_scatter4(n_pad, e_pad)
    scatter128 = _make_scatter128(n_pad, e_pad)

    att = None
    for l in range(N_LAYERS):
        q, k, v = _proj_qkv(xp, p['Wq'][l], p['Wk'][l], p['Wv'][l], bm_n)
        e = _proj_e(ea, p['We'][l], bm_e)
        k_e = k[src] + e
        v_e = v[src] + e
        logits = (q[dst].reshape(e_pad, HEADS, HEAD_DIM)
                  * k_e.reshape(e_pad, HEADS, HEAD_DIM)).sum(-1) * INV_SQRT_D
        ex = jnp.exp(jnp.minimum(logits, LOGIT_CLAMP))
        s2 = scatter4(ex, dst, z4)
        rec = _srecip(s2.reshape(2, n_pad, HEADS), n_pad)
        wmsg = jnp.repeat(ex, HEAD_DIM, axis=1) * v_e
        msgh = jnp.concatenate([wmsg[:, :EMB // 2], wmsg[:, EMB // 2:]],
                               axis=0)
        agg2 = scatter128(msgh, dst, z128)
        if l == N_LAYERS - 1:
            att = (ex * rec[dst]).mean(axis=1)
        xp = _update(agg2.reshape(2, n_pad, EMB // 2), rec, xp,
                     p['Wo'][l], p['Ws'][l], bm_n)
    pooled = _pool(xp[:n_real], starts, n_real)
    return pooled, att[:e_real]


def kernel(x_raw, edge_attr_raw, x_tree, edge_attr_tree, params,
           edge_index_raw, batch_raw, edge_index_tree, batch_tree):
    gids = jnp.arange(N_GRAPHS + 1, dtype=jnp.int32)
    starts_r = jnp.searchsorted(batch_raw, gids, side='left').astype(jnp.int32)
    starts_t = jnp.searchsorted(batch_tree, gids, side='left').astype(jnp.int32)

    pooled_r, att_r = _encoder(x_raw, edge_attr_raw, edge_index_raw, starts_r,
                               params['enc_raw'], N_RAW, 10240, 163840,
                               1024, 4096)
    pooled_t, att_t = _encoder(x_tree, edge_attr_tree, edge_index_tree,
                               starts_t, params['enc_tree'], N_TREE, 4096,
                               8192, 1024, 4096)
    y = _head(pooled_t, pooled_r, params)
    return (y, edge_index_raw, att_r, edge_index_tree, att_t)
